# baseline (speedup 1.0000x reference)
import functools

import jax
import jax.numpy as jnp
from jax import lax
from jax.experimental import pallas as pl
from jax.experimental.pallas import tpu as pltpu
from jax.experimental.pallas import tpu_sc as plsc

_F = 26
_D = 16
_CHUNK = 128
_NW = 32
_L = 16


def _sc_retile(emb_t):
    d, n = emb_t.shape
    n_full = n // 128
    per_w = (n_full + _NW - 1) // _NW
    fire_ahead = 8

    mesh = plsc.VectorSubcoreMesh(core_axis_name="c", subcore_axis_name="s")

    @functools.partial(
        pl.kernel,
        out_type=jax.ShapeDtypeStruct((n_full * 16, 128), jnp.float32),
        mesh=mesh,
        scratch_types=[pltpu.SemaphoreType.DMA],
        compiler_params=pltpu.CompilerParams(use_tc_tiling_on_sc=True),
    )
    def retile_kernel(src_hbm, out_hbm, sem):
        wid = lax.axis_index("s") * 2 + lax.axis_index("c")

        def issue(tc):
            for g in range(2):
                pltpu.async_copy(
                    src_hbm.at[pl.ds(g * 8, 8), pl.ds(tc * 128, 128)],
                    out_hbm.at[pl.ds(tc * 16 + g * 8, 8), :],
                    sem,
                )

        def drain():
            for g in range(2):
                pltpu.make_async_copy(
                    src_hbm.at[pl.ds(g * 8, 8), pl.ds(0, 128)],
                    out_hbm.at[pl.ds(g * 8, 8), :],
                    sem,
                ).wait()

        def body(k, carry):
            tc = k * _NW + wid

            @pl.when(tc < n_full)
            def _():
                issue(tc)

            @pl.when(k >= fire_ahead)
            def _():
                t_old = (k - fire_ahead) * _NW + wid
                @pl.when(t_old < n_full)
                def _():
                    drain()

            return carry

        lax.fori_loop(0, per_w + fire_ahead, body, 0, unroll=False)

    return retile_kernel(emb_t)


def _sc_transpose(blocks, tail_rows, n, d):
    n_full = n // 128
    tail = n - n_full * 128
    per_w = (n_full + _NW - 1) // _NW

    mesh = plsc.VectorSubcoreMesh(core_axis_name="c", subcore_axis_name="s")

    @functools.partial(
        pl.kernel,
        out_type=jax.ShapeDtypeStruct((n * d,), jnp.float32),
        mesh=mesh,
        scratch_types=[
            pltpu.VMEM((2048,), jnp.float32),
            pltpu.VMEM((2048,), jnp.float32),
            pltpu.SemaphoreType.DMA,
        ],
        compiler_params=pltpu.CompilerParams(needs_layout_passes=False),
    )
    def tr_kernel(src_hbm, tail_hbm, out_hbm, buf, obuf, sem):
        wid = lax.axis_index("s") * 2 + lax.axis_index("c")
        lanes_d = jnp.arange(16, dtype=jnp.int32) * d

        def do_block(t):
            pltpu.sync_copy(src_hbm.at[pl.ds(t * 2048, 2048)], buf)
            for dd in range(d):
                for j in range(128 // _L):
                    v = buf[pl.ds(dd * 128 + j * _L, _L)]
                    plsc.store_scatter(obuf, [lanes_d + (j * _L * d + dd)], v)
            pltpu.sync_copy(obuf, out_hbm.at[pl.ds(t * 128 * d, 128 * d)])

        def body(k, carry):
            t = k * _NW + wid

            @pl.when(t < n_full)
            def _():
                do_block(t)

            return carry

        lax.fori_loop(0, per_w, body, 0, unroll=False)

        if tail:
            @pl.when(wid == _NW - 1)
            def _():
                pltpu.sync_copy(tail_hbm,
                                out_hbm.at[pl.ds(n_full * 128 * d, tail * d)])

    return tr_kernel(blocks, tail_rows)


def _sc_fm(x_chunks, emb_table, w1_flat, batch):
    num_chunks = batch // _CHUNK
    chunks_per_w = num_chunks // _NW

    mesh = plsc.VectorSubcoreMesh(core_axis_name="c", subcore_axis_name="s")

    @functools.partial(
        pl.kernel,
        out_type=jax.ShapeDtypeStruct((num_chunks, _CHUNK, _D), jnp.float32),
        mesh=mesh,
        scratch_types=[
            pltpu.VMEM((_F, _CHUNK), jnp.int32),
            pltpu.VMEM((_F * _CHUNK, _D), jnp.float32),
            pltpu.VMEM((_F, _CHUNK), jnp.float32),
            pltpu.VMEM((_CHUNK + _L, ), jnp.float32),
            pltpu.VMEM((_CHUNK, _D), jnp.float32),
            pltpu.SemaphoreType.DMA,
        ],
        compiler_params=pltpu.CompilerParams(use_tc_tiling_on_sc=False),
    )
    def fm_kernel(x_hbm, emb_hbm, w1_hbm, out_hbm,
                  idx_v, rows_v, w1_v, lin_v, h_v, sem):
        wid = lax.axis_index("s") * 2 + lax.axis_index("c")

        for c in range(chunks_per_w):
            g = wid * chunks_per_w + c
            pltpu.sync_copy(x_hbm.at[g], idx_v)

            descs = []
            for f in range(_F):
                descs.append(pltpu.async_copy(
                    emb_hbm.at[idx_v.at[f]],
                    rows_v.at[pl.ds(f * _CHUNK, _CHUNK), :],
                    sem,
                ))
                descs.append(pltpu.async_copy(
                    w1_hbm.at[idx_v.at[f]],
                    w1_v.at[f],
                    sem,
                ))
            for dsc in descs:
                dsc.wait()

            for k in range(_CHUNK // _L):
                acc = w1_v[0, pl.ds(k * _L, _L)]
                for f in range(1, _F):
                    acc = acc + w1_v[f, pl.ds(k * _L, _L)]
                lin_v[pl.ds(k * _L, _L)] = acc

            def body(b, carry):
                v = rows_v[b]
                s = v
                sq = v * v
                for f in range(1, _F):
                    v = rows_v[f * _CHUNK + b]
                    s = s + v
                    sq = sq + v * v
                lin = lin_v[pl.ds(b, _L)][0]
                h_v[b] = 0.5 * (s * s - sq) + lin
                return carry

            lax.fori_loop(0, _CHUNK, body, 0, unroll=False)
            pltpu.sync_copy(h_v, out_hbm.at[g])

    return fm_kernel(x_chunks, emb_table, w1_flat)


def _tc_mlp(h, W0, b0, W1, b1, W2, b2):
    batch = h.shape[0]

    def mlp_kernel(h_ref, w0_ref, b0_ref, w1_ref, b1_ref, w2_ref, b2_ref, o_ref):
        z = h_ref[...]
        z = jnp.maximum(
            jnp.dot(z, w0_ref[...], preferred_element_type=jnp.float32)
            + b0_ref[...], 0.0)
        z = jnp.maximum(
            jnp.dot(z, w1_ref[...], preferred_element_type=jnp.float32)
            + b1_ref[...], 0.0)
        out = jnp.sum(z * w2_ref[...], axis=1) + b2_ref[0, 0]
        o_ref[...] = jax.nn.sigmoid(out)

    return pl.pallas_call(
        mlp_kernel,
        out_shape=jax.ShapeDtypeStruct((batch,), jnp.float32),
    )(h, W0, b0.reshape(1, -1), W1, b1.reshape(1, -1), W2.reshape(1, -1),
      b2.reshape(1, 1))


def kernel(inputs, emb_table, w1_table, W0, b0, W1, b1, W2, b2):
    batch, ncols = inputs.shape
    half = ncols // 2
    x = inputs[:, half:]
    x_chunks = x.reshape(batch // _CHUNK, _CHUNK, _F).transpose(0, 2, 1)
    n_rows = emb_table.shape[0]
    blocks = _sc_retile(emb_table.T)
    tail_rows = emb_table[(n_rows // 128) * 128:].reshape(-1)
    emb_lin = _sc_transpose(blocks.reshape(-1), tail_rows, n_rows,
                            _D).reshape(n_rows, _D)
    h = _sc_fm(x_chunks, emb_lin, w1_table.reshape(-1), batch)
    return _tc_mlp(h.reshape(batch, _D), W0, b0, W1, b1, W2, b2)


# speedup vs baseline: 8.7663x; 8.7663x over previous
import functools

import jax
import jax.numpy as jnp
from jax import lax
from jax.experimental import pallas as pl
from jax.experimental.pallas import tpu as pltpu
from jax.experimental.pallas import tpu_sc as plsc

_F = 26
_D = 16
_CHUNK = 128
_NW = 32
_L = 16


def _sc_linearize(emb_t, tail_rows, n, d):
    n_full = n // 128
    tail = n - n_full * 128
    per_w = (n_full + _NW - 1) // _NW
    outer = (per_w + 1) // 2

    mesh = plsc.VectorSubcoreMesh(core_axis_name="c", subcore_axis_name="s")

    @functools.partial(
        pl.kernel,
        out_type=jax.ShapeDtypeStruct((n * d,), jnp.float32),
        mesh=mesh,
        scratch_types=[
            pltpu.VMEM((d, 128), jnp.float32),
            pltpu.VMEM((d, 128), jnp.float32),
            pltpu.VMEM((128 * _D,), jnp.float32),
            pltpu.VMEM((128 * _D,), jnp.float32),
            pltpu.SemaphoreType.DMA,
            pltpu.SemaphoreType.DMA,
            pltpu.SemaphoreType.DMA,
            pltpu.SemaphoreType.DMA,
        ],
        compiler_params=pltpu.CompilerParams(use_tc_tiling_on_sc=True,
                                             needs_layout_passes=False),
    )
    def lin_kernel(src_hbm, tail_hbm, out_hbm,
                   buf0, buf1, obuf0, obuf1, si0, si1, so0, so1):
        wid = lax.axis_index("s") * 2 + lax.axis_index("c")
        bufs = (buf0, buf1)
        obufs = (obuf0, obuf1)
        sis = (si0, si1)
        sos = (so0, so1)
        lanes_d = jnp.arange(16, dtype=jnp.int32) * d

        def tcol(k):
            return k * _NW + wid

        def issue_load(k, b):
            @pl.when(tcol(k) < n_full)
            def _():
                pltpu.async_copy(
                    src_hbm.at[:, pl.ds(tcol(k) * 128, 128)], bufs[b], sis[b])

        def wait_load(b):
            pltpu.make_async_copy(
                src_hbm.at[:, pl.ds(0, 128)], bufs[b], sis[b]).wait()

        def wait_store(b):
            pltpu.make_async_copy(
                out_hbm.at[pl.ds(0, 128 * _D)], obufs[b], sos[b]).wait()

        issue_load(0, 0)
        issue_load(1, 1)

        def body(k2, carry):
            for b in range(2):
                k = k2 * 2 + b

                @pl.when(tcol(k) < n_full)
                def _(k=k, b=b):
                    wait_load(b)

                    @pl.when(k >= 2)
                    def _():
                        wait_store(b)

                    ob = obufs[b]
                    bf = bufs[b]
                    for dd in range(d):
                        for j in range(128 // _L):
                            v = bf[dd, pl.ds(j * _L, _L)]
                            plsc.store_scatter(
                                ob, [lanes_d + (j * _L * d + dd)], v)
                    pltpu.async_copy(
                        ob, out_hbm.at[pl.ds(tcol(k) * 128 * d, 128 * d)],
                        sos[b])
                    issue_load(k + 2, b)

            return carry

        lax.fori_loop(0, outer, body, 0, unroll=False)

        wait_store(0)
        wait_store(1)

        if tail:
            @pl.when(wid == _NW - 1)
            def _():
                pltpu.sync_copy(
                    tail_hbm, out_hbm.at[pl.ds(n_full * 128 * d, tail * d)])

    return lin_kernel(emb_t, tail_rows)


def _sc_fm(x_chunks, emb_table, w1_flat, batch):
    num_chunks = batch // _CHUNK
    chunks_per_w = num_chunks // _NW

    mesh = plsc.VectorSubcoreMesh(core_axis_name="c", subcore_axis_name="s")

    @functools.partial(
        pl.kernel,
        out_type=jax.ShapeDtypeStruct((num_chunks, _CHUNK, _D), jnp.float32),
        mesh=mesh,
        scratch_types=[
            pltpu.VMEM((_F, _CHUNK), jnp.int32),
            pltpu.VMEM((_F * _CHUNK, _D), jnp.float32),
            pltpu.VMEM((_F, _CHUNK), jnp.float32),
            pltpu.VMEM((_CHUNK + _L, ), jnp.float32),
            pltpu.VMEM((_CHUNK, _D), jnp.float32),
            pltpu.SemaphoreType.DMA,
        ],
        compiler_params=pltpu.CompilerParams(use_tc_tiling_on_sc=False),
    )
    def fm_kernel(x_hbm, emb_hbm, w1_hbm, out_hbm,
                  idx_v, rows_v, w1_v, lin_v, h_v, sem):
        wid = lax.axis_index("s") * 2 + lax.axis_index("c")

        for c in range(chunks_per_w):
            g = wid * chunks_per_w + c
            pltpu.sync_copy(x_hbm.at[g], idx_v)

            descs = []
            for f in range(_F):
                descs.append(pltpu.async_copy(
                    emb_hbm.at[idx_v.at[f]],
                    rows_v.at[pl.ds(f * _CHUNK, _CHUNK), :],
                    sem,
                ))
                descs.append(pltpu.async_copy(
                    w1_hbm.at[idx_v.at[f]],
                    w1_v.at[f],
                    sem,
                ))
            for dsc in descs:
                dsc.wait()

            for k in range(_CHUNK // _L):
                acc = w1_v[0, pl.ds(k * _L, _L)]
                for f in range(1, _F):
                    acc = acc + w1_v[f, pl.ds(k * _L, _L)]
                lin_v[pl.ds(k * _L, _L)] = acc

            def body(b, carry):
                v = rows_v[b]
                s = v
                sq = v * v
                for f in range(1, _F):
                    v = rows_v[f * _CHUNK + b]
                    s = s + v
                    sq = sq + v * v
                lin = lin_v[pl.ds(b, _L)][0]
                h_v[b] = 0.5 * (s * s - sq) + lin
                return carry

            lax.fori_loop(0, _CHUNK, body, 0, unroll=False)
            pltpu.sync_copy(h_v, out_hbm.at[g])

    return fm_kernel(x_chunks, emb_table, w1_flat)


def _tc_mlp(h, W0, b0, W1, b1, W2, b2):
    batch = h.shape[0]

    def mlp_kernel(h_ref, w0_ref, b0_ref, w1_ref, b1_ref, w2_ref, b2_ref, o_ref):
        z = h_ref[...]
        z = jnp.maximum(
            jnp.dot(z, w0_ref[...], preferred_element_type=jnp.float32)
            + b0_ref[...], 0.0)
        z = jnp.maximum(
            jnp.dot(z, w1_ref[...], preferred_element_type=jnp.float32)
            + b1_ref[...], 0.0)
        out = jnp.sum(z * w2_ref[...], axis=1) + b2_ref[0, 0]
        o_ref[...] = jax.nn.sigmoid(out)

    return pl.pallas_call(
        mlp_kernel,
        out_shape=jax.ShapeDtypeStruct((batch,), jnp.float32),
    )(h, W0, b0.reshape(1, -1), W1, b1.reshape(1, -1), W2.reshape(1, -1),
      b2.reshape(1, 1))


def kernel(inputs, emb_table, w1_table, W0, b0, W1, b1, W2, b2):
    batch, ncols = inputs.shape
    half = ncols // 2
    x = inputs[:, half:]
    x_chunks = x.reshape(batch // _CHUNK, _CHUNK, _F).transpose(0, 2, 1)
    n_rows = emb_table.shape[0]
    tail_rows = emb_table[(n_rows // 128) * 128:].reshape(-1)
    emb_lin = _sc_linearize(emb_table.T, tail_rows, n_rows,
                            _D).reshape(n_rows, _D)
    h = _sc_fm(x_chunks, emb_lin, w1_table.reshape(-1), batch)
    return _tc_mlp(h.reshape(batch, _D), W0, b0, W1, b1, W2, b2)


# speedup vs baseline: 8.8924x; 1.0144x over previous
import functools

import jax
import jax.numpy as jnp
from jax import lax
from jax.experimental import pallas as pl
from jax.experimental.pallas import tpu as pltpu
from jax.experimental.pallas import tpu_sc as plsc

_F = 26
_D = 16
_CHUNK = 128
_NW = 32
_L = 16


def _sc_linearize(emb_t, tail_rows, n, d):
    n_full = n // 128
    tail = n - n_full * 128
    _W = 512
    n_grp = n_full * 128 // _W
    per_w = (n_grp + _NW - 1) // _NW
    outer = (per_w + 1) // 2

    mesh = plsc.VectorSubcoreMesh(core_axis_name="c", subcore_axis_name="s")

    @functools.partial(
        pl.kernel,
        out_type=jax.ShapeDtypeStruct((n * d,), jnp.float32),
        mesh=mesh,
        scratch_types=[
            pltpu.VMEM((d, _W), jnp.float32),
            pltpu.VMEM((d, _W), jnp.float32),
            pltpu.VMEM((_W * _D,), jnp.float32),
            pltpu.VMEM((_W * _D,), jnp.float32),
            pltpu.SemaphoreType.DMA,
            pltpu.SemaphoreType.DMA,
            pltpu.SemaphoreType.DMA,
            pltpu.SemaphoreType.DMA,
        ],
        compiler_params=pltpu.CompilerParams(use_tc_tiling_on_sc=True,
                                             needs_layout_passes=False),
    )
    def lin_kernel(src_hbm, tail_hbm, out_hbm,
                   buf0, buf1, obuf0, obuf1, si0, si1, so0, so1):
        wid = lax.axis_index("s") * 2 + lax.axis_index("c")
        bufs = (buf0, buf1)
        obufs = (obuf0, obuf1)
        sis = (si0, si1)
        sos = (so0, so1)
        lanes_d = jnp.arange(16, dtype=jnp.int32) * d

        def grp(k):
            return k * _NW + wid

        def issue_load(k, b):
            @pl.when(grp(k) < n_grp)
            def _():
                pltpu.async_copy(
                    src_hbm.at[:, pl.ds(grp(k) * _W, _W)], bufs[b], sis[b])

        def wait_load(b):
            pltpu.make_async_copy(
                src_hbm.at[:, pl.ds(0, _W)], bufs[b], sis[b]).wait()

        def wait_store(b):
            pltpu.make_async_copy(
                out_hbm.at[pl.ds(0, _W * _D)], obufs[b], sos[b]).wait()

        issue_load(0, 0)
        issue_load(1, 1)

        def body(k2, carry):
            for b in range(2):
                k = k2 * 2 + b

                @pl.when(grp(k) < n_grp)
                def _(k=k, b=b):
                    wait_load(b)

                    @pl.when(k >= 2)
                    def _():
                        wait_store(b)

                    ob = obufs[b]
                    bf = bufs[b]
                    for dd in range(d):
                        for j in range(_W // _L):
                            v = bf[dd, pl.ds(j * _L, _L)]
                            plsc.store_scatter(
                                ob, [lanes_d + (j * _L * d + dd)], v)
                    pltpu.async_copy(
                        ob, out_hbm.at[pl.ds(grp(k) * _W * d, _W * d)],
                        sos[b])
                    issue_load(k + 2, b)

            return carry

        lax.fori_loop(0, outer, body, 0, unroll=False)

        wait_store(0)
        wait_store(1)

        if tail:
            @pl.when(wid == _NW - 1)
            def _():
                pltpu.sync_copy(
                    tail_hbm, out_hbm.at[pl.ds(n_full * 128 * d, tail * d)])

    return lin_kernel(emb_t, tail_rows)


def _sc_fm(x_chunks, emb_table, w1_flat, batch):
    num_chunks = batch // _CHUNK
    chunks_per_w = num_chunks // _NW

    mesh = plsc.VectorSubcoreMesh(core_axis_name="c", subcore_axis_name="s")

    @functools.partial(
        pl.kernel,
        out_type=jax.ShapeDtypeStruct((num_chunks, _CHUNK, _D), jnp.float32),
        mesh=mesh,
        scratch_types=[
            pltpu.VMEM((_F, _CHUNK), jnp.int32),
            pltpu.VMEM((_F * _CHUNK, _D), jnp.float32),
            pltpu.VMEM((_F, _CHUNK), jnp.float32),
            pltpu.VMEM((_CHUNK + _L, ), jnp.float32),
            pltpu.VMEM((_CHUNK, _D), jnp.float32),
            pltpu.SemaphoreType.DMA,
        ],
        compiler_params=pltpu.CompilerParams(use_tc_tiling_on_sc=False),
    )
    def fm_kernel(x_hbm, emb_hbm, w1_hbm, out_hbm,
                  idx_v, rows_v, w1_v, lin_v, h_v, sem):
        wid = lax.axis_index("s") * 2 + lax.axis_index("c")

        for c in range(chunks_per_w):
            g = wid * chunks_per_w + c
            pltpu.sync_copy(x_hbm.at[g], idx_v)

            descs = []
            for f in range(_F):
                descs.append(pltpu.async_copy(
                    emb_hbm.at[idx_v.at[f]],
                    rows_v.at[pl.ds(f * _CHUNK, _CHUNK), :],
                    sem,
                ))
                descs.append(pltpu.async_copy(
                    w1_hbm.at[idx_v.at[f]],
                    w1_v.at[f],
                    sem,
                ))
            for dsc in descs:
                dsc.wait()

            for k in range(_CHUNK // _L):
                acc = w1_v[0, pl.ds(k * _L, _L)]
                for f in range(1, _F):
                    acc = acc + w1_v[f, pl.ds(k * _L, _L)]
                lin_v[pl.ds(k * _L, _L)] = acc

            def body(b, carry):
                v = rows_v[b]
                s = v
                sq = v * v
                for f in range(1, _F):
                    v = rows_v[f * _CHUNK + b]
                    s = s + v
                    sq = sq + v * v
                lin = lin_v[pl.ds(b, _L)][0]
                h_v[b] = 0.5 * (s * s - sq) + lin
                return carry

            lax.fori_loop(0, _CHUNK, body, 0, unroll=False)
            pltpu.sync_copy(h_v, out_hbm.at[g])

    return fm_kernel(x_chunks, emb_table, w1_flat)


def _tc_mlp(h, W0, b0, W1, b1, W2, b2):
    batch = h.shape[0]

    def mlp_kernel(h_ref, w0_ref, b0_ref, w1_ref, b1_ref, w2_ref, b2_ref, o_ref):
        z = h_ref[...]
        z = jnp.maximum(
            jnp.dot(z, w0_ref[...], preferred_element_type=jnp.float32)
            + b0_ref[...], 0.0)
        z = jnp.maximum(
            jnp.dot(z, w1_ref[...], preferred_element_type=jnp.float32)
            + b1_ref[...], 0.0)
        out = jnp.sum(z * w2_ref[...], axis=1) + b2_ref[0, 0]
        o_ref[...] = jax.nn.sigmoid(out)

    return pl.pallas_call(
        mlp_kernel,
        out_shape=jax.ShapeDtypeStruct((batch,), jnp.float32),
    )(h, W0, b0.reshape(1, -1), W1, b1.reshape(1, -1), W2.reshape(1, -1),
      b2.reshape(1, 1))


def kernel(inputs, emb_table, w1_table, W0, b0, W1, b1, W2, b2):
    batch, ncols = inputs.shape
    half = ncols // 2
    x = inputs[:, half:]
    x_chunks = x.reshape(batch // _CHUNK, _CHUNK, _F).transpose(0, 2, 1)
    n_rows = emb_table.shape[0]
    tail_rows = emb_table[(n_rows // 128) * 128:].reshape(-1)
    emb_lin = _sc_linearize(emb_table.T, tail_rows, n_rows,
                            _D).reshape(n_rows, _D)
    h = _sc_fm(x_chunks, emb_lin, w1_table.reshape(-1), batch)
    return _tc_mlp(h.reshape(batch, _D), W0, b0, W1, b1, W2, b2)


# speedup vs baseline: 10.3860x; 1.1680x over previous
import functools

import jax
import jax.numpy as jnp
from jax import lax
from jax.experimental import pallas as pl
from jax.experimental.pallas import tpu as pltpu
from jax.experimental.pallas import tpu_sc as plsc

_F = 26
_D = 16
_CHUNK = 128
_NW = 32
_L = 16


def _sc_linearize(emb_t, tail_rows, n, d):
    n_full = n // 128
    tail = n - n_full * 128
    _W = 512
    n_grp = n_full * 128 // _W
    per_w = (n_grp + _NW - 1) // _NW
    outer = (per_w + 1) // 2

    mesh = plsc.VectorSubcoreMesh(core_axis_name="c", subcore_axis_name="s")

    @functools.partial(
        pl.kernel,
        out_type=jax.ShapeDtypeStruct((n * d,), jnp.float32),
        mesh=mesh,
        scratch_types=[
            pltpu.VMEM((d, _W), jnp.float32),
            pltpu.VMEM((d, _W), jnp.float32),
            pltpu.VMEM((_W * _D,), jnp.float32),
            pltpu.VMEM((_W * _D,), jnp.float32),
            pltpu.SemaphoreType.DMA,
            pltpu.SemaphoreType.DMA,
            pltpu.SemaphoreType.DMA,
            pltpu.SemaphoreType.DMA,
        ],
        compiler_params=pltpu.CompilerParams(use_tc_tiling_on_sc=True,
                                             needs_layout_passes=False),
    )
    def lin_kernel(src_hbm, tail_hbm, out_hbm,
                   buf0, buf1, obuf0, obuf1, si0, si1, so0, so1):
        wid = lax.axis_index("s") * 2 + lax.axis_index("c")
        bufs = (buf0, buf1)
        obufs = (obuf0, obuf1)
        sis = (si0, si1)
        sos = (so0, so1)
        lanes_d = jnp.arange(16, dtype=jnp.int32) * d

        def grp(k):
            return k * _NW + wid

        def issue_load(k, b):
            @pl.when(grp(k) < n_grp)
            def _():
                pltpu.async_copy(
                    src_hbm.at[:, pl.ds(grp(k) * _W, _W)], bufs[b], sis[b])

        def wait_load(b):
            pltpu.make_async_copy(
                src_hbm.at[:, pl.ds(0, _W)], bufs[b], sis[b]).wait()

        def wait_store(b):
            pltpu.make_async_copy(
                out_hbm.at[pl.ds(0, _W * _D)], obufs[b], sos[b]).wait()

        issue_load(0, 0)
        issue_load(1, 1)

        def body(k2, carry):
            for b in range(2):
                k = k2 * 2 + b

                @pl.when(grp(k) < n_grp)
                def _(k=k, b=b):
                    wait_load(b)

                    @pl.when(k >= 2)
                    def _():
                        wait_store(b)

                    ob = obufs[b]
                    bf = bufs[b]
                    for dd in range(d):
                        for j in range(_W // _L):
                            v = bf[dd, pl.ds(j * _L, _L)]
                            plsc.store_scatter(
                                ob, [lanes_d + (j * _L * d + dd)], v)
                    pltpu.async_copy(
                        ob, out_hbm.at[pl.ds(grp(k) * _W * d, _W * d)],
                        sos[b])
                    issue_load(k + 2, b)

            return carry

        lax.fori_loop(0, outer, body, 0, unroll=False)

        wait_store(0)
        wait_store(1)

        if tail:
            @pl.when(wid == _NW - 1)
            def _():
                pltpu.sync_copy(
                    tail_hbm, out_hbm.at[pl.ds(n_full * 128 * d, tail * d)])

    return lin_kernel(emb_t, tail_rows)


def _sc_fm(x_chunks, emb_table, w1_flat, batch):
    num_chunks = batch // _CHUNK
    nc = num_chunks // _NW

    mesh = plsc.VectorSubcoreMesh(core_axis_name="c", subcore_axis_name="s")

    @functools.partial(
        pl.kernel,
        out_type=jax.ShapeDtypeStruct((num_chunks, _CHUNK, _D), jnp.float32),
        mesh=mesh,
        scratch_types=[
            [pltpu.VMEM((_F, _CHUNK), jnp.int32) for _ in range(2)],
            [pltpu.VMEM((_F * _CHUNK, _D), jnp.float32) for _ in range(2)],
            [pltpu.VMEM((_F, _CHUNK), jnp.float32) for _ in range(2)],
            [pltpu.VMEM((_CHUNK + _L, ), jnp.float32) for _ in range(2)],
            [pltpu.VMEM((_CHUNK, _D), jnp.float32) for _ in range(2)],
            [pltpu.SemaphoreType.DMA for _ in range(5)],
        ],
        compiler_params=pltpu.CompilerParams(use_tc_tiling_on_sc=False),
    )
    def fm_kernel(x_hbm, emb_hbm, w1_hbm, out_hbm,
                  idx_v, rows_v, w1_v, lin_v, h_v, sems):
        wid = lax.axis_index("s") * 2 + lax.axis_index("c")
        s_idx, s_g0, s_g1, s_h0, s_h1 = sems
        s_g = (s_g0, s_g1)
        s_h = (s_h0, s_h1)
        g_descs = [None, None]
        h_descs = [None, None]

        def load_idx(c, b):
            return pltpu.async_copy(x_hbm.at[wid * nc + c], idx_v[b], s_idx)

        def fire_gathers(b):
            descs = []
            for f in range(_F):
                descs.append(pltpu.async_copy(
                    emb_hbm.at[idx_v[b].at[f]],
                    rows_v[b].at[pl.ds(f * _CHUNK, _CHUNK), :],
                    s_g[b],
                ))
                descs.append(pltpu.async_copy(
                    w1_hbm.at[idx_v[b].at[f]],
                    w1_v[b].at[f],
                    s_g[b],
                ))
            g_descs[b] = descs

        def compute_and_store(c, b):
            for dsc in g_descs[b]:
                dsc.wait()
            if h_descs[b] is not None:
                h_descs[b].wait()

            for k in range(_CHUNK // _L):
                acc = w1_v[b][0, pl.ds(k * _L, _L)]
                for f in range(1, _F):
                    acc = acc + w1_v[b][f, pl.ds(k * _L, _L)]
                lin_v[b][pl.ds(k * _L, _L)] = acc

            def body(r, carry):
                v = rows_v[b][r]
                s = v
                sq = v * v
                for f in range(1, _F):
                    v = rows_v[b][f * _CHUNK + r]
                    s = s + v
                    sq = sq + v * v
                lin = lin_v[b][pl.ds(r, _L)][0]
                h_v[b][r] = 0.5 * (s * s - sq) + lin
                return carry

            lax.fori_loop(0, _CHUNK, body, 0, unroll=False)
            h_descs[b] = pltpu.async_copy(
                h_v[b], out_hbm.at[wid * nc + c], s_h[b])

        idx_descs = [load_idx(0, 0)]
        for c in range(nc):
            b = c % 2
            idx_descs[c].wait()
            fire_gathers(b)
            if c + 1 < nc:
                if c >= 1:
                    compute_and_store(c - 1, 1 - b)
                idx_descs.append(load_idx(c + 1, 1 - b))
        compute_and_store(nc - 1, (nc - 1) % 2)
        for dsc in h_descs:
            if dsc is not None:
                dsc.wait()

    return fm_kernel(x_chunks, emb_table, w1_flat)


def _tc_mlp(h, W0, b0, W1, b1, W2, b2):
    batch = h.shape[0]

    def mlp_kernel(h_ref, w0_ref, b0_ref, w1_ref, b1_ref, w2_ref, b2_ref, o_ref):
        z = h_ref[...]
        z = jnp.maximum(
            jnp.dot(z, w0_ref[...], preferred_element_type=jnp.float32)
            + b0_ref[...], 0.0)
        z = jnp.maximum(
            jnp.dot(z, w1_ref[...], preferred_element_type=jnp.float32)
            + b1_ref[...], 0.0)
        out = jnp.sum(z * w2_ref[...], axis=1) + b2_ref[0, 0]
        o_ref[...] = jax.nn.sigmoid(out)

    return pl.pallas_call(
        mlp_kernel,
        out_shape=jax.ShapeDtypeStruct((batch,), jnp.float32),
    )(h, W0, b0.reshape(1, -1), W1, b1.reshape(1, -1), W2.reshape(1, -1),
      b2.reshape(1, 1))


def kernel(inputs, emb_table, w1_table, W0, b0, W1, b1, W2, b2):
    batch, ncols = inputs.shape
    half = ncols // 2
    x = inputs[:, half:]
    x_chunks = x.reshape(batch // _CHUNK, _CHUNK, _F).transpose(0, 2, 1)
    n_rows = emb_table.shape[0]
    tail_rows = emb_table[(n_rows // 128) * 128:].reshape(-1)
    emb_lin = _sc_linearize(emb_table.T, tail_rows, n_rows,
                            _D).reshape(n_rows, _D)
    h = _sc_fm(x_chunks, emb_lin, w1_table.reshape(-1), batch)
    return _tc_mlp(h.reshape(batch, _D), W0, b0, W1, b1, W2, b2)
